# baseline (speedup 1.0000x reference)
import functools

import jax
import jax.numpy as jnp
from jax import lax
from jax.experimental import pallas as pl
from jax.experimental.pallas import tpu as pltpu
from jax.experimental.pallas import tpu_sc as plsc

B, S, T = 4, 2048, 1024
N = B * S
NC, NS, L = 2, 16, 16
NW = NC * NS
ROWS_PER_W = N // NW
G = 16
NG = ROWS_PER_W // G
NSEG = 8
SEG = T // NSEG
CHUNKS = T // L

_mesh = plsc.VectorSubcoreMesh(core_axis_name="c", subcore_axis_name="s")


@functools.partial(
    pl.kernel,
    mesh=_mesh,
    out_type=jax.ShapeDtypeStruct((N, T), jnp.float32),
    scratch_types=[
        pltpu.VMEM((G * T,), jnp.float32),
        pltpu.VMEM((G * T,), jnp.float32),
        pltpu.VMEM((G * T,), jnp.float32),
        pltpu.VMEM((G, T), jnp.float32),
        pltpu.VMEM((G, T), jnp.float32),
        pltpu.VMEM((G,), jnp.int32),
        pltpu.VMEM((G,), jnp.int32),
        pltpu.SemaphoreType.DMA,
        pltpu.SemaphoreType.DMA,
        pltpu.SemaphoreType.DMA,
    ],
    compiler_params=pltpu.CompilerParams(needs_layout_passes=False),
)
def _crf_head(x_hbm, t_hbm, out_hbm, xb0, xb1, xb2, tb0, tb1, ib0, ib1,
              in_sem, g_sem, out_sem):
    xb = (xb0, xb1, xb2)
    tb = (tb0, tb1)
    ib = (ib0, ib1)
    wid = lax.axis_index("s") * NC + lax.axis_index("c")
    base = wid * ROWS_PER_W
    lane = lax.iota(jnp.int32, L)

    def start_in(g):
        x_v = xb[g % 3]
        return [
            pltpu.async_copy(x_hbm.at[base + g * G + r],
                             x_v.at[pl.ds(r * T, T)], in_sem)
            for r in range(G)
        ]

    def argmax(g):
        x_v = xb[g % 3]

        def body(j, carry):
            out = []
            for k in range(NSEG):
                m, bc, av = carry[k]
                vals = plsc.load_gather(x_v, [av])
                cmp = vals > m
                m = jnp.where(cmp, vals, m)
                bc = jnp.where(cmp, av, bc)
                out.append((m, bc, av + 1))
            return tuple(out)

        init = tuple(
            (jnp.full((L,), -jnp.inf, jnp.float32),
             lane * T + (k * SEG),
             lane * T + (k * SEG))
            for k in range(NSEG))
        fin = lax.fori_loop(0, SEG, body, init, unroll=2)
        m, bc, _ = fin[0]
        for k in range(1, NSEG):
            mk, bck, _ = fin[k]
            cmp = mk > m
            m = jnp.where(cmp, mk, m)
            bc = jnp.where(cmp, bck, bc)
        ib[g % 2][...] = bc & (T - 1)

    def start_gather(g):
        return pltpu.async_copy(t_hbm.at[ib[g % 2]], tb[g % 2], g_sem)

    def add(g):
        x_v, t_v = xb[g % 3], tb[g % 2]

        def body(c, _):
            off = c * L
            vals = [t_v[r, pl.ds(off, L)] for r in range(G)]
            for r in range(G):
                plsc.addupdate(x_v.at[pl.ds(r * T + off, L)], vals[r])
            return 0

        lax.fori_loop(0, CHUNKS, body, 0)

    def start_out(g):
        x_v = xb[g % 3]
        return [
            pltpu.async_copy(x_v.at[pl.ds(r * T, T)],
                             out_hbm.at[base + g * G + r], out_sem)
            for r in range(G)
        ]

    def wait_all(handles):
        for h in handles:
            h.wait()

    ins = {0: start_in(0), 1: start_in(1)}
    gathers = {}
    outs = {}
    wait_all(ins[0])
    argmax(0)
    gathers[0] = start_gather(0)
    for g in range(NG):
        if g + 2 < NG:
            if g >= 1:
                wait_all(outs[g - 1])
            ins[g + 2] = start_in(g + 2)
        if g + 1 < NG:
            wait_all(ins[g + 1])
            argmax(g + 1)
            gathers[g + 1] = start_gather(g + 1)
        gathers[g].wait()
        add(g)
        outs[g] = start_out(g)
    wait_all(outs[NG - 2])
    wait_all(outs[NG - 1])


def kernel(launch_matrix, transitions):
    x = launch_matrix.reshape(N, T)
    out = _crf_head(x, transitions)
    return out.reshape(B, S, T)


# speedup vs baseline: 1.7373x; 1.7373x over previous
import functools

import jax
import jax.numpy as jnp
from jax import lax
from jax.experimental import pallas as pl
from jax.experimental.pallas import tpu as pltpu
from jax.experimental.pallas import tpu_sc as plsc

B, S, T = 4, 2048, 1024
N = B * S
NC, NS, L = 2, 16, 16
NW = NC * NS
ROWS_PER_W = N // NW
G = 16
NG = ROWS_PER_W // G
NB = 4
NACC = 2
CHUNKS = T // L
CPA = CHUNKS // NACC

_mesh = plsc.VectorSubcoreMesh(core_axis_name="c", subcore_axis_name="s")


@functools.partial(
    pl.kernel,
    mesh=_mesh,
    out_type=jax.ShapeDtypeStruct((N, T), jnp.float32),
    scratch_types=[
        pltpu.VMEM((G, T), jnp.float32),
        pltpu.VMEM((G, T), jnp.float32),
        pltpu.VMEM((G, T), jnp.float32),
        pltpu.VMEM((G, T), jnp.float32),
        pltpu.VMEM((G, T), jnp.float32),
        pltpu.VMEM((G, T), jnp.float32),
        pltpu.VMEM((G,), jnp.int32),
        pltpu.VMEM((G,), jnp.int32),
        pltpu.SemaphoreType.DMA,
        pltpu.SemaphoreType.DMA,
        pltpu.SemaphoreType.DMA,
    ],
    compiler_params=pltpu.CompilerParams(needs_layout_passes=False),
)
def _crf_head(x_hbm, t_hbm, out_hbm, xb0, xb1, xb2, xb3, tb0, tb1,
              ib0, ib1, in_sem, g_sem, out_sem):
    xb = (xb0, xb1, xb2, xb3)
    tb = (tb0, tb1)
    ib = (ib0, ib1)
    wid = lax.axis_index("s") * NC + lax.axis_index("c")
    base = wid * ROWS_PER_W
    lane = lax.iota(jnp.int32, L)

    def start_in(g, b):
        pltpu.async_copy(x_hbm.at[pl.ds(base + g * G, G)], xb[b], in_sem)

    def wait_in(b):
        pltpu.make_async_copy(x_hbm.at[pl.ds(0, G)], xb[b], in_sem).wait()

    def argmax(b, i):
        x_v = xb[b]
        ivec = jnp.zeros((L,), jnp.int32)
        for r in range(G):
            def body(c, carry):
                out = []
                for a in range(NACC):
                    m, bch, cnt = carry[a]
                    v = x_v[r, pl.ds((a * CPA + c) * L, L)]
                    cmp = v > m
                    m = jnp.where(cmp, v, m)
                    bch = jnp.where(cmp, cnt, bch)
                    out.append((m, bch, cnt + 1))
                return tuple(out)

            init = tuple(
                (jnp.full((L,), -jnp.inf, jnp.float32),
                 jnp.full((L,), a * CPA, jnp.int32),
                 jnp.full((L,), a * CPA, jnp.int32))
                for a in range(NACC))
            fin = lax.fori_loop(0, CPA, body, init)
            m, bch, _ = fin[0]
            for a in range(1, NACC):
                ma, bca, _ = fin[a]
                cmp = ma > m
                m = jnp.where(cmp, ma, m)
                bch = jnp.where(cmp, bca, bch)
            ms = jnp.max(m)
            col = (bch << 4) + lane
            cand = jnp.where(m == jnp.full((L,), ms), col,
                             jnp.full((L,), T, jnp.int32))
            cmin = jnp.min(cand)
            ivec = jnp.where(lane == r, jnp.full((L,), cmin), ivec)
        ib[i][...] = ivec

    def start_gather(i):
        pltpu.async_copy(t_hbm.at[ib[i]], tb[i], g_sem)

    def wait_gather(i):
        pltpu.make_async_copy(t_hbm.at[ib[i]], tb[i], g_sem).wait()

    def add(b, i):
        x_v, t_v = xb[b], tb[i]

        def body(c, _):
            off = c * L
            for r in range(G):
                plsc.addupdate(x_v.at[r, pl.ds(off, L)], t_v[r, pl.ds(off, L)])
            return 0

        lax.fori_loop(0, CHUNKS, body, 0)

    def start_out(g, b):
        pltpu.async_copy(xb[b], out_hbm.at[pl.ds(base + g * G, G)], out_sem)

    def wait_out(b):
        pltpu.make_async_copy(xb[b], out_hbm.at[pl.ds(0, G)], out_sem).wait()

    start_in(0, 0)
    start_in(1, 1)
    start_in(2, 2)
    wait_in(0)
    argmax(0, 0)
    start_gather(0)

    def outer(o, carry):
        for b in range(NB):
            g = o * NB + b
            i = b % 2

            @pl.when(g + 1 < NG)
            def _():
                wait_in((b + 1) % NB)
                argmax((b + 1) % NB, (i + 1) % 2)
                start_gather((i + 1) % 2)

            @pl.when(jnp.logical_and(g >= 1, g + 3 < NG))
            def _():
                wait_out((b + 3) % NB)

            @pl.when(g + 3 < NG)
            def _():
                start_in(g + 3, (b + 3) % NB)

            wait_gather(i)
            add(b, i)
            start_out(g, b)
        return carry

    lax.fori_loop(0, NG // NB, outer, 0)
    for b in range(NB):
        wait_out(b)


def kernel(launch_matrix, transitions):
    x = launch_matrix.reshape(N, T)
    out = _crf_head(x, transitions)
    return out.reshape(B, S, T)


# speedup vs baseline: 1.9340x; 1.1132x over previous
import functools

import jax
import jax.numpy as jnp
from jax import lax
from jax.experimental import pallas as pl
from jax.experimental.pallas import tpu as pltpu
from jax.experimental.pallas import tpu_sc as plsc

B, S, T = 4, 2048, 1024
N = B * S
NC, NS, L = 2, 16, 16
NW = NC * NS
ROWS_PER_W = N // NW
G = 16
NG = ROWS_PER_W // G
NB = 4
NACC = 4
CHUNKS = T // L
CPA = CHUNKS // NACC

_mesh = plsc.VectorSubcoreMesh(core_axis_name="c", subcore_axis_name="s")


@functools.partial(
    pl.kernel,
    mesh=_mesh,
    out_type=jax.ShapeDtypeStruct((N, T), jnp.float32),
    scratch_types=[
        pltpu.VMEM((G, T), jnp.float32),
        pltpu.VMEM((G, T), jnp.float32),
        pltpu.VMEM((G, T), jnp.float32),
        pltpu.VMEM((G, T), jnp.float32),
        pltpu.VMEM((G, T), jnp.float32),
        pltpu.VMEM((G, T), jnp.float32),
        pltpu.VMEM((G,), jnp.int32),
        pltpu.VMEM((G,), jnp.int32),
        pltpu.SemaphoreType.DMA,
        pltpu.SemaphoreType.DMA,
        pltpu.SemaphoreType.DMA,
    ],
    compiler_params=pltpu.CompilerParams(needs_layout_passes=False),
)
def _crf_head(x_hbm, t_hbm, out_hbm, xb0, xb1, xb2, xb3, tb0, tb1,
              ib0, ib1, in_sem, g_sem, out_sem):
    xb = (xb0, xb1, xb2, xb3)
    tb = (tb0, tb1)
    ib = (ib0, ib1)
    wid = lax.axis_index("s") * NC + lax.axis_index("c")
    base = wid * ROWS_PER_W
    lane = lax.iota(jnp.int32, L)

    def start_in(g, b):
        pltpu.async_copy(x_hbm.at[pl.ds(base + g * G, G)], xb[b], in_sem)

    def wait_in(b):
        pltpu.make_async_copy(x_hbm.at[pl.ds(0, G)], xb[b], in_sem).wait()

    def argmax(b, i):
        x_v = xb[b]

        def row_body(r, ivec):
            m = [jnp.full((L,), -jnp.inf, jnp.float32)] * NACC
            bch = [jnp.zeros((L,), jnp.int32)] * NACC
            for c in range(CPA):
                for a in range(NACC):
                    ch = a * CPA + c
                    v = x_v[r, pl.ds(ch * L, L)]
                    cmp = v > m[a]
                    m[a] = jnp.where(cmp, v, m[a])
                    bch[a] = jnp.where(cmp, jnp.full((L,), ch, jnp.int32),
                                       bch[a])
            mm, bb = m[0], bch[0]
            for a in range(1, NACC):
                cmp = m[a] > mm
                mm = jnp.where(cmp, m[a], mm)
                bb = jnp.where(cmp, bch[a], bb)
            ms = jnp.max(mm)
            col = (bb << 4) + lane
            cand = jnp.where(mm == jnp.full((L,), ms), col,
                             jnp.full((L,), T, jnp.int32))
            cmin = jnp.min(cand)
            return jnp.where(lane == r, jnp.full((L,), cmin), ivec)

        ib[i][...] = lax.fori_loop(0, G, row_body,
                                   jnp.zeros((L,), jnp.int32))

    def start_gather(i):
        pltpu.async_copy(t_hbm.at[ib[i]], tb[i], g_sem)

    def wait_gather(i):
        pltpu.make_async_copy(t_hbm.at[ib[i]], tb[i], g_sem).wait()

    def add(b, i):
        x_v, t_v = xb[b], tb[i]

        def row_body(r, carry):
            for c in range(CHUNKS):
                off = c * L
                plsc.addupdate(x_v.at[r, pl.ds(off, L)], t_v[r, pl.ds(off, L)])
            return carry

        lax.fori_loop(0, G, row_body, 0)

    def start_out(g, b):
        pltpu.async_copy(xb[b], out_hbm.at[pl.ds(base + g * G, G)], out_sem)

    def wait_out(b):
        pltpu.make_async_copy(xb[b], out_hbm.at[pl.ds(0, G)], out_sem).wait()

    start_in(0, 0)
    start_in(1, 1)
    start_in(2, 2)
    wait_in(0)
    argmax(0, 0)
    start_gather(0)

    def outer(o, carry):
        for b in range(NB):
            g = o * NB + b
            i = b % 2

            @pl.when(g + 1 < NG)
            def _():
                wait_in((b + 1) % NB)
                argmax((b + 1) % NB, (i + 1) % 2)
                start_gather((i + 1) % 2)

            @pl.when(jnp.logical_and(g >= 1, g + 3 < NG))
            def _():
                wait_out((b + 3) % NB)

            @pl.when(g + 3 < NG)
            def _():
                start_in(g + 3, (b + 3) % NB)

            wait_gather(i)
            add(b, i)
            start_out(g, b)
        return carry

    lax.fori_loop(0, NG // NB, outer, 0)
    for b in range(NB):
        wait_out(b)


def kernel(launch_matrix, transitions):
    x = launch_matrix.reshape(N, T)
    out = _crf_head(x, transitions)
    return out.reshape(B, S, T)
